# baseline (speedup 1.0000x reference)
import functools

import jax
import jax.numpy as jnp
from jax import lax
from jax.experimental import pallas as pl
from jax.experimental.pallas import tpu as pltpu
from jax.experimental.pallas import tpu_sc as plsc

NUM_USERS = 100000
NUM_ITEMS = 100000
EMBED_DIM = 64
BATCH = 16384

_info = plsc.get_sparse_core_info()
_NC, _NS = _info.num_cores, _info.num_subcores
_NW = _NC * _NS
_BPW = BATCH // _NW


def _gather_kernel(user_hbm, item_hbm, uid_hbm, iid_hbm, out_u, out_i,
                   uidx_v, iidx_v, urows_v, irows_v, sem_u, sem_i):
    wid = lax.axis_index("s") * _NC + lax.axis_index("c")
    base = wid * _BPW
    pltpu.sync_copy(uid_hbm.at[pl.ds(base, _BPW)], uidx_v)
    pltpu.sync_copy(iid_hbm.at[pl.ds(base, _BPW)], iidx_v)
    cp_u = pltpu.async_copy(user_hbm.at[uidx_v], urows_v, sem_u)
    cp_i = pltpu.async_copy(item_hbm.at[iidx_v], irows_v, sem_i)
    cp_u.wait()
    cp_i.wait()
    pltpu.sync_copy(urows_v, out_u.at[pl.ds(base, _BPW)])
    pltpu.sync_copy(irows_v, out_i.at[pl.ds(base, _BPW)])


@jax.jit
def kernel(user_embedding, item_embedding, user_ids, item_ids):
    mesh = plsc.VectorSubcoreMesh(core_axis_name="c", subcore_axis_name="s")
    fn = functools.partial(
        pl.kernel,
        mesh=mesh,
        compiler_params=pltpu.CompilerParams(use_tc_tiling_on_sc=False),
        out_type=(
            jax.ShapeDtypeStruct((BATCH, EMBED_DIM), jnp.float32),
            jax.ShapeDtypeStruct((BATCH, EMBED_DIM), jnp.float32),
        ),
        scratch_types=[
            pltpu.VMEM((_BPW,), jnp.int32),
            pltpu.VMEM((_BPW,), jnp.int32),
            pltpu.VMEM((_BPW, EMBED_DIM), jnp.float32),
            pltpu.VMEM((_BPW, EMBED_DIM), jnp.float32),
            pltpu.SemaphoreType.DMA,
            pltpu.SemaphoreType.DMA,
        ],
    )(_gather_kernel)
    return fn(user_embedding, item_embedding,
              user_ids.astype(jnp.int32), item_ids.astype(jnp.int32))


# speedup vs baseline: 3.0783x; 3.0783x over previous
import functools

import jax
import jax.numpy as jnp
from jax import lax
from jax.experimental import pallas as pl
from jax.experimental.pallas import tpu as pltpu
from jax.experimental.pallas import tpu_sc as plsc

NUM_ROWS = 100000
EMBED_DIM = 64
BATCH = 16384

_FEATS_PER_SUB = EMBED_DIM // 16
_HALF = BATCH // 2


def _gather_table(tref, idx_v, oref, fb, row_v, out_v):
    for j in range(_FEATS_PER_SUB):
        d = fb * _FEATS_PER_SUB + j
        pltpu.sync_copy(tref.at[d], row_v)
        for h in range(2):
            @plsc.parallel_loop(0, _HALF, step=16, unroll=8)
            def _(i):
                ids = idx_v[pl.ds(h * _HALF + i, 16)]
                out_v[pl.ds(i, 16)] = plsc.load_gather(row_v, [ids])
            pltpu.sync_copy(out_v, oref.at[d, pl.ds(h * _HALF, _HALF)])


def _emb_kernel(ut, it, uid_hbm, iid_hbm, out_u, out_i, idx_v, row_v, out_v):
    core = lax.axis_index("c")
    fb = lax.axis_index("s")

    @pl.when(core == 0)
    def _():
        pltpu.sync_copy(uid_hbm, idx_v)
        _gather_table(ut, idx_v, out_u, fb, row_v, out_v)

    @pl.when(core == 1)
    def _():
        pltpu.sync_copy(iid_hbm, idx_v)
        _gather_table(it, idx_v, out_i, fb, row_v, out_v)


@jax.jit
def kernel(user_embedding, item_embedding, user_ids, item_ids):
    mesh = plsc.VectorSubcoreMesh(core_axis_name="c", subcore_axis_name="s")
    fn = functools.partial(
        pl.kernel,
        mesh=mesh,
        compiler_params=pltpu.CompilerParams(needs_layout_passes=False),
        out_type=(
            jax.ShapeDtypeStruct((EMBED_DIM, BATCH), jnp.float32),
            jax.ShapeDtypeStruct((EMBED_DIM, BATCH), jnp.float32),
        ),
        scratch_types=[
            pltpu.VMEM((BATCH,), jnp.int32),
            pltpu.VMEM((NUM_ROWS,), jnp.float32),
            pltpu.VMEM((_HALF,), jnp.float32),
        ],
    )(_emb_kernel)
    out_ut, out_it = fn(user_embedding.T, item_embedding.T,
                        user_ids.astype(jnp.int32), item_ids.astype(jnp.int32))
    return (out_ut.T, out_it.T)


# speedup vs baseline: 3.1426x; 1.0209x over previous
import functools

import jax
import jax.numpy as jnp
from jax import lax
from jax.experimental import pallas as pl
from jax.experimental.pallas import tpu as pltpu
from jax.experimental.pallas import tpu_sc as plsc

NUM_ROWS = 100000
EMBED_DIM = 64
BATCH = 16384

_FEATS_PER_SUB = EMBED_DIM // 16
_HALF = BATCH // 2


_CHUNK = 4096
_NCHUNK = BATCH // _CHUNK


def _gather_table(tref, idxref, oref, fb, idx_v, row_v, out_a, out_b, sems):
    cp_idx = pltpu.async_copy(idxref, idx_v, sems[2])
    cp_row = pltpu.async_copy(tref.at[fb * _FEATS_PER_SUB], row_v, sems[3])
    cp_idx.wait()
    cp_row.wait()
    bufs = (out_a, out_b)
    pending = [None, None]
    seg = 0
    for j in range(_FEATS_PER_SUB):
        d = fb * _FEATS_PER_SUB + j
        if j > 0:
            pltpu.sync_copy(tref.at[d], row_v)
        for q in range(_NCHUNK):
            b = seg % 2
            if pending[b] is not None:
                pending[b].wait()
            buf = bufs[b]

            @plsc.parallel_loop(0, _CHUNK, step=16, unroll=8)
            def _(i, q=q, buf=buf):
                ids = idx_v[pl.ds(q * _CHUNK + i, 16)]
                buf[pl.ds(i, 16)] = plsc.load_gather(row_v, [ids])

            pending[b] = pltpu.async_copy(
                buf, oref.at[d, pl.ds(q * _CHUNK, _CHUNK)], sems[b])
            seg += 1
    pending[0].wait()
    pending[1].wait()


def _emb_kernel(ut, it, uid_hbm, iid_hbm, out_u, out_i,
                idx_v, row_v, out_a, out_b, s0, s1, s2, s3):
    core = lax.axis_index("c")
    fb = lax.axis_index("s")
    sems = (s0, s1, s2, s3)

    @pl.when(core == 0)
    def _():
        _gather_table(ut, uid_hbm, out_u, fb, idx_v, row_v, out_a, out_b, sems)

    @pl.when(core == 1)
    def _():
        _gather_table(it, iid_hbm, out_i, fb, idx_v, row_v, out_a, out_b, sems)


@jax.jit
def kernel(user_embedding, item_embedding, user_ids, item_ids):
    mesh = plsc.VectorSubcoreMesh(core_axis_name="c", subcore_axis_name="s")
    fn = functools.partial(
        pl.kernel,
        mesh=mesh,
        compiler_params=pltpu.CompilerParams(needs_layout_passes=False),
        out_type=(
            jax.ShapeDtypeStruct((EMBED_DIM, BATCH), jnp.float32),
            jax.ShapeDtypeStruct((EMBED_DIM, BATCH), jnp.float32),
        ),
        scratch_types=[
            pltpu.VMEM((BATCH,), jnp.int32),
            pltpu.VMEM((NUM_ROWS,), jnp.float32),
            pltpu.VMEM((_CHUNK,), jnp.float32),
            pltpu.VMEM((_CHUNK,), jnp.float32),
            pltpu.SemaphoreType.DMA,
            pltpu.SemaphoreType.DMA,
            pltpu.SemaphoreType.DMA,
            pltpu.SemaphoreType.DMA,
        ],
    )(_emb_kernel)
    out_ut, out_it = fn(user_embedding.T, item_embedding.T,
                        user_ids.astype(jnp.int32), item_ids.astype(jnp.int32))
    return (out_ut.T, out_it.T)
